# baseline (speedup 1.0000x reference)
import functools

import jax
import jax.numpy as jnp
from jax import lax
from jax.experimental import pallas as pl
from jax.experimental.pallas import tpu as pltpu
from jax.experimental.pallas import tpu_sc as plsc

_N = 1048576
_NC = 2
_NS = 16
_NW = _NC * _NS
_BPW = _N // _NW
_LANES = 16
_P = 4
_CPP = _BPW // _P

_mesh = plsc.VectorSubcoreMesh(core_axis_name="c", subcore_axis_name="s")


@functools.partial(
    pl.kernel,
    out_type=jax.ShapeDtypeStruct((_N,), jnp.float32),
    mesh=_mesh,
    compiler_params=pltpu.CompilerParams(needs_layout_passes=False),
    scratch_types=[
        pltpu.VMEM((_BPW,), jnp.int32),
        pltpu.VMEM((_BPW,), jnp.int32),
        pltpu.VMEM((_BPW,), jnp.float32),
        pltpu.VMEM((18, 3), jnp.float32),
        [pltpu.SemaphoreType.DMA] * (3 * _P + 1),
    ],
)
def _gather_kernel(z_hbm, q_hbm, tab_hbm, out_hbm, z_v, q_v, o_v, tab_v, sems):
    wid = lax.axis_index("s") * _NC + lax.axis_index("c")
    base = wid * _BPW

    tab_cp = pltpu.async_copy(tab_hbm, tab_v, sems[3 * _P])
    in_cps = []
    for p in range(_P):
        off = p * _CPP
        zc = pltpu.async_copy(z_hbm.at[pl.ds(base + off, _CPP)],
                              z_v.at[pl.ds(off, _CPP)], sems[p])
        qc = pltpu.async_copy(q_hbm.at[pl.ds(base + off, _CPP)],
                              q_v.at[pl.ds(off, _CPP)], sems[_P + p])
        in_cps.append((zc, qc))
    tab_cp.wait()

    out_cps = []
    for p in range(_P):
        off = p * _CPP
        zc, qc = in_cps[p]
        zc.wait()
        qc.wait()

        @plsc.parallel_loop(off, off + _CPP, step=_LANES, unroll=16)
        def _body(i):
            zz = z_v[pl.ds(i, _LANES)]
            qq = q_v[pl.ds(i, _LANES)]
            o_v[pl.ds(i, _LANES)] = plsc.load_gather(tab_v, [zz, qq])

        out_cps.append(
            pltpu.async_copy(o_v.at[pl.ds(off, _CPP)],
                             out_hbm.at[pl.ds(base + off, _CPP)],
                             sems[2 * _P + p]))
    for cp in out_cps:
        cp.wait()


def kernel(z, charge, energy_table):
    return _gather_kernel(z, charge, energy_table)


# speedup vs baseline: 1.2115x; 1.2115x over previous
import functools

import jax
import jax.numpy as jnp
from jax import lax
from jax.experimental import pallas as pl
from jax.experimental.pallas import tpu as pltpu
from jax.experimental.pallas import tpu_sc as plsc

_N = 1048576
_NC = 2
_NS = 16
_NW = _NC * _NS
_BPW = _N // _NW
_LANES = 16
_P = 4
_CPP = _BPW // _P

_mesh = plsc.VectorSubcoreMesh(core_axis_name="c", subcore_axis_name="s")


@functools.partial(
    pl.kernel,
    out_type=jax.ShapeDtypeStruct((_N,), jnp.float32),
    mesh=_mesh,
    compiler_params=pltpu.CompilerParams(needs_layout_passes=False),
    scratch_types=[
        pltpu.VMEM((_BPW,), jnp.int32),
        pltpu.VMEM((_BPW,), jnp.int32),
        pltpu.VMEM((_BPW,), jnp.float32),
        pltpu.VMEM((64,), jnp.float32),
        [pltpu.SemaphoreType.DMA] * (3 * _P + 1),
    ],
)
def _gather_kernel(z_hbm, q_hbm, tab_hbm, out_hbm, z_v, q_v, o_v, tab_v, sems):
    wid = lax.axis_index("s") * _NC + lax.axis_index("c")
    base = wid * _BPW

    tab_cp = pltpu.async_copy(tab_hbm, tab_v, sems[3 * _P])
    in_cps = []
    for p in range(_P):
        off = p * _CPP
        zc = pltpu.async_copy(z_hbm.at[pl.ds(base + off, _CPP)],
                              z_v.at[pl.ds(off, _CPP)], sems[p])
        qc = pltpu.async_copy(q_hbm.at[pl.ds(base + off, _CPP)],
                              q_v.at[pl.ds(off, _CPP)], sems[_P + p])
        in_cps.append((zc, qc))
    tab_cp.wait()

    out_cps = []
    for p in range(_P):
        off = p * _CPP
        zc, qc = in_cps[p]
        zc.wait()
        qc.wait()

        @plsc.parallel_loop(off, off + _CPP, step=_LANES, unroll=16)
        def _body(i):
            idx = z_v[pl.ds(i, _LANES)] * 3 + q_v[pl.ds(i, _LANES)]
            o_v[pl.ds(i, _LANES)] = plsc.load_gather(tab_v, [idx])

        out_cps.append(
            pltpu.async_copy(o_v.at[pl.ds(off, _CPP)],
                             out_hbm.at[pl.ds(base + off, _CPP)],
                             sems[2 * _P + p]))
    for cp in out_cps:
        cp.wait()


def kernel(z, charge, energy_table):
    tab = jnp.pad(energy_table.reshape(-1), (0, 64 - energy_table.size))
    return _gather_kernel(z, charge, tab)
